# baseline (speedup 1.0000x reference)
import functools

import jax
import jax.numpy as jnp
import numpy as np
from jax import lax
from jax.experimental import pallas as pl
from jax.experimental.pallas import tpu as pltpu
from jax.experimental.pallas import tpu_sc as plsc

NC = 2
NS = 16
NW = NC * NS
LANES = 16
CHUNK = 112
NSLICE = 11
BM2 = 2048

_PA = np.arange(64).reshape(4, 16)
_PA = (_PA // 16 * 32 + _PA % 16).reshape(-1)
_PB = _PA + 16


def _round_bf16_word(a_f32, b_f32):
    ua = lax.bitcast_convert_type(a_f32, jnp.int32)
    ub = lax.bitcast_convert_type(b_f32, jnp.int32)
    ra = ua + (jnp.int32(0x7FFF) + ((ua >> 16) & 1))
    rb = ub + (jnp.int32(0x7FFF) + ((ub >> 16) & 1))
    return ((ra >> 16) & jnp.int32(0xFFFF)) | ((rb >> 16) << 16)


def _sc_gather_bf16(table, idx3d, hidden):
    chunks_per_w = idx3d.shape[1]
    assert chunks_per_w % 2 == 0 and chunks_per_w >= 4
    b_per_w = chunks_per_w * CHUNK
    rows = NW * b_per_w
    half = CHUNK // 2
    groups = hidden // (2 * LANES)
    mesh = plsc.VectorSubcoreMesh(core_axis_name="c", subcore_axis_name="s")

    @functools.partial(
        pl.kernel,
        mesh=mesh,
        out_type=jax.ShapeDtypeStruct((rows // 2, hidden), jnp.int32),
        scratch_types=[
            pltpu.VMEM((chunks_per_w, CHUNK), jnp.int32),
            pltpu.VMEM((CHUNK, hidden), jnp.float32),
            pltpu.VMEM((CHUNK, hidden), jnp.float32),
            pltpu.VMEM((half, hidden), jnp.int32),
            pltpu.VMEM((half, hidden), jnp.int32),
            pltpu.SemaphoreType.DMA,
            pltpu.SemaphoreType.DMA,
            pltpu.SemaphoreType.DMA,
            pltpu.SemaphoreType.DMA,
        ],
    )
    def gather_k(table_hbm, idx_hbm, out_hbm, idx_v, f0, f1, b0, b1,
                 g0, g1, o0, o1):
        wid = lax.axis_index("s") * NC + lax.axis_index("c")
        base = wid * (b_per_w // 2)
        pltpu.sync_copy(idx_hbm.at[wid], idx_v)

        def gather(j, fbuf, sem):
            pltpu.async_copy(table_hbm.at[idx_v.at[j]], fbuf, sem)

        def gather_wait(fbuf, sem):
            pltpu.make_async_copy(table_hbm.at[idx_v.at[0]], fbuf, sem).wait()

        def convert(fbuf, bbuf):
            def crow(r, carry):
                for h in range(2):
                    for g in range(groups):
                        a = fbuf[2 * r + h, pl.ds(32 * g, LANES)]
                        b = fbuf[2 * r + h, pl.ds(32 * g + LANES, LANES)]
                        bbuf[r, pl.ds((hidden // 2) * h + LANES * g, LANES)] = (
                            _round_bf16_word(a, b))
                return carry
            lax.fori_loop(0, half, crow, 0)

        def put(j, bbuf, sem):
            pltpu.async_copy(
                bbuf, out_hbm.at[pl.ds(base + j * half, half)], sem)

        def put_wait(bbuf, sem):
            pltpu.make_async_copy(
                bbuf, out_hbm.at[pl.ds(base, half)], sem).wait()

        gather(0, f0, g0)
        gather(1, f1, g1)

        gather_wait(f0, g0)
        convert(f0, b0)
        put(0, b0, o0)
        gather(2, f0, g0)
        gather_wait(f1, g1)
        convert(f1, b1)
        put(1, b1, o1)
        gather(3, f1, g1)

        def body(i, carry):
            j = 2 * i
            put_wait(b0, o0)
            gather_wait(f0, g0)
            convert(f0, b0)
            put(j, b0, o0)
            gather(j + 2, f0, g0)
            put_wait(b1, o1)
            gather_wait(f1, g1)
            convert(f1, b1)
            put(j + 1, b1, o1)
            gather(j + 3, f1, g1)
            return carry

        lax.fori_loop(1, chunks_per_w // 2 - 1, body, 0)

        j = chunks_per_w - 2
        put_wait(b0, o0)
        gather_wait(f0, g0)
        convert(f0, b0)
        put(j, b0, o0)
        put_wait(b1, o1)
        gather_wait(f1, g1)
        convert(f1, b1)
        put(j + 1, b1, o1)
        put_wait(b0, o0)
        put_wait(b1, o1)

    return gather_k(table, idx3d)


def _tc_linear_slice(x_slice, pos_a, pos_b, wa, wb, bias2, out_prev,
                     block0, pos0):
    n2, hidden = x_slice.shape
    n2_total = out_prev.shape[0] if out_prev is not None else None
    nb = n2 // BM2

    def body(x_ref, pa_ref, pb_ref, wa_ref, wb_ref, b_ref, *rest):
        o_ref = rest[-1]
        w32 = x_ref[...]
        xa = lax.bitcast_convert_type(w32 << 16, jnp.float32)
        xb = lax.bitcast_convert_type((w32 >> 16) << 16, jnp.float32)
        o_ref[...] = (
            jnp.dot(xa + pa_ref[0], wa_ref[...],
                    preferred_element_type=jnp.float32)
            + jnp.dot(xb + pb_ref[0], wb_ref[...],
                      preferred_element_type=jnp.float32)
            + b_ref[...]
        )

    in_specs = [
        pl.BlockSpec((BM2, hidden), lambda i: (i, 0)),
        pl.BlockSpec((1, 1, hidden), lambda i: (pos0 + i, 0, 0)),
        pl.BlockSpec((1, 1, hidden), lambda i: (pos0 + i, 0, 0)),
        pl.BlockSpec((hidden, 2 * hidden), lambda i: (0, 0)),
        pl.BlockSpec((hidden, 2 * hidden), lambda i: (0, 0)),
        pl.BlockSpec((1, 2 * hidden), lambda i: (0, 0)),
    ]
    args = [x_slice, pos_a, pos_b, wa, wb, bias2]
    io_aliases = {}
    if out_prev is not None:
        in_specs.append(pl.BlockSpec(memory_space=pl.ANY))
        args.append(out_prev)
        io_aliases = {6: 0}
        total = n2_total
    else:
        total = nb * BM2 * NSLICE

    return pl.pallas_call(
        body,
        grid=(nb,),
        in_specs=in_specs,
        out_specs=pl.BlockSpec((BM2, 2 * hidden), lambda i: (block0 + i, 0)),
        out_shape=jax.ShapeDtypeStruct((total, 2 * hidden), jnp.float32),
        input_output_aliases=io_aliases,
    )(*args)


def kernel(caption, token_embedding, positional_embedding, W, b):
    batch, seq = caption.shape
    vocab, hidden = token_embedding.shape
    b_total = batch * seq

    idx = caption.astype(jnp.int32).T.reshape(-1)
    n_chunks = b_total // CHUNK
    chunks_per_w = n_chunks // NW
    cw_slice = chunks_per_w // NSLICE
    rows_slice = NW * cw_slice * CHUNK
    idx4d = idx.reshape(NSLICE, NW, cw_slice, CHUNK)

    pa = jnp.asarray(_PA)
    pb = jnp.asarray(_PB)
    wt = W.T
    za = jnp.zeros((hidden // 2, hidden), wt.dtype)
    wa = jnp.concatenate([
        jnp.concatenate([wt[pa], za], axis=1),
        jnp.concatenate([za, wt[pa]], axis=1),
    ], axis=0)
    wb = jnp.concatenate([
        jnp.concatenate([wt[pb], za], axis=1),
        jnp.concatenate([za, wt[pb]], axis=1),
    ], axis=0)
    pos = positional_embedding[:seq]
    pos_a = jnp.concatenate([pos[:, pa], pos[:, pa]], axis=1)
    pos_a = pos_a.reshape(seq, 1, hidden)
    pos_b = jnp.concatenate([pos[:, pb], pos[:, pb]], axis=1)
    pos_b = pos_b.reshape(seq, 1, hidden)
    bias2 = jnp.concatenate([b, b]).reshape(1, 2 * hidden)

    blocks_slice = rows_slice // (2 * BM2)
    out = None
    for k in range(NSLICE):
        packed_k = _sc_gather_bf16(token_embedding, idx4d[k], hidden)
        out = _tc_linear_slice(packed_k, pos_a, pos_b, wa, wb, bias2, out,
                               k * blocks_slice, k * blocks_slice)

    return out.reshape(seq, batch, hidden).transpose(1, 0, 2)


# speedup vs baseline: 1.0897x; 1.0897x over previous
import functools

import jax
import jax.numpy as jnp
import numpy as np
from jax import lax
from jax.experimental import pallas as pl
from jax.experimental.pallas import tpu as pltpu
from jax.experimental.pallas import tpu_sc as plsc

NC = 2
NS = 16
NW = NC * NS
LANES = 16
CHUNK = 112
NSLICE = 11
BM2 = 2048

_PA = np.arange(64).reshape(4, 16)
_PA = (_PA // 16 * 32 + _PA % 16).reshape(-1)
_PB = _PA + 16


def _round_bf16_word(a_f32, b_f32):
    ua = lax.bitcast_convert_type(a_f32, jnp.int32)
    ub = lax.bitcast_convert_type(b_f32, jnp.int32)
    ra = ua + jnp.int32(0x8000)
    rb = ub + jnp.int32(0x8000)
    return ((ra >> 16) & jnp.int32(0xFFFF)) | ((rb >> 16) << 16)


def _sc_gather_bf16(table, idx3d, hidden):
    chunks_per_w = idx3d.shape[1]
    nbuf = min(4, chunks_per_w)
    b_per_w = chunks_per_w * CHUNK
    rows = NW * b_per_w
    half = CHUNK // 2
    groups = hidden // (2 * LANES)
    mesh = plsc.VectorSubcoreMesh(core_axis_name="c", subcore_axis_name="s")

    @functools.partial(
        pl.kernel,
        mesh=mesh,
        out_type=jax.ShapeDtypeStruct((rows // 2, hidden), jnp.int32),
        scratch_types=(
            [pltpu.VMEM((chunks_per_w, CHUNK), jnp.int32)]
            + [pltpu.VMEM((CHUNK, hidden), jnp.float32)] * nbuf
            + [pltpu.VMEM((half, hidden), jnp.int32)] * nbuf
            + [pltpu.SemaphoreType.DMA] * (2 * nbuf)
        ),
    )
    def gather_k(table_hbm, idx_hbm, out_hbm, idx_v, *bufs):
        fb = bufs[:nbuf]
        bb = bufs[nbuf:2 * nbuf]
        gs = bufs[2 * nbuf:3 * nbuf]
        os = bufs[3 * nbuf:]
        wid = lax.axis_index("s") * NC + lax.axis_index("c")
        base = wid * (b_per_w // 2)
        pltpu.sync_copy(idx_hbm.at[wid], idx_v)

        def gather(j, p):
            pltpu.async_copy(table_hbm.at[idx_v.at[j]], fb[p], gs[p])

        def gather_wait(p):
            pltpu.make_async_copy(
                table_hbm.at[idx_v.at[0]], fb[p], gs[p]).wait()

        def convert(p):
            fbuf, bbuf = fb[p], bb[p]

            def crow(r, carry):
                for h in range(2):
                    for g in range(groups):
                        a = fbuf[2 * r + h, pl.ds(32 * g, LANES)]
                        b = fbuf[2 * r + h, pl.ds(32 * g + LANES, LANES)]
                        bbuf[r, pl.ds((hidden // 2) * h + LANES * g, LANES)] = (
                            _round_bf16_word(a, b))
                return carry
            lax.fori_loop(0, half, crow, 0)

        def put(j, p):
            pltpu.async_copy(
                bb[p], out_hbm.at[pl.ds(base + j * half, half)], os[p])

        def put_wait(p):
            pltpu.make_async_copy(
                bb[p], out_hbm.at[pl.ds(base, half)], os[p]).wait()

        for j in range(nbuf):
            gather(j, j)
        for j in range(chunks_per_w):
            p = j % nbuf
            if j >= nbuf:
                put_wait(p)
            gather_wait(p)
            convert(p)
            put(j, p)
            if j + nbuf < chunks_per_w:
                gather(j + nbuf, p)
        for p in range(nbuf):
            put_wait(p)

    return gather_k(table, idx3d)


def _tc_linear_slice(x_slice, pos_a, pos_b, wa, wb, bias2, out_prev,
                     block0, pos0):
    n2, hidden = x_slice.shape
    n2_total = out_prev.shape[0] if out_prev is not None else None
    nb = n2 // BM2

    def body(x_ref, pa_ref, pb_ref, wa_ref, wb_ref, b_ref, *rest):
        o_ref = rest[-1]
        w32 = x_ref[...]
        xa = lax.bitcast_convert_type(w32 << 16, jnp.float32)
        xb = lax.bitcast_convert_type((w32 >> 16) << 16, jnp.float32)
        o_ref[...] = (
            jnp.dot(xa + pa_ref[0], wa_ref[...],
                    preferred_element_type=jnp.float32)
            + jnp.dot(xb + pb_ref[0], wb_ref[...],
                      preferred_element_type=jnp.float32)
            + b_ref[...]
        )

    in_specs = [
        pl.BlockSpec((BM2, hidden), lambda i: (i, 0)),
        pl.BlockSpec((1, 1, hidden), lambda i: (pos0 + i, 0, 0)),
        pl.BlockSpec((1, 1, hidden), lambda i: (pos0 + i, 0, 0)),
        pl.BlockSpec((hidden, 2 * hidden), lambda i: (0, 0)),
        pl.BlockSpec((hidden, 2 * hidden), lambda i: (0, 0)),
        pl.BlockSpec((1, 2 * hidden), lambda i: (0, 0)),
    ]
    args = [x_slice, pos_a, pos_b, wa, wb, bias2]
    io_aliases = {}
    if out_prev is not None:
        in_specs.append(pl.BlockSpec(memory_space=pl.ANY))
        args.append(out_prev)
        io_aliases = {6: 0}
        total = n2_total
    else:
        total = nb * BM2 * NSLICE

    return pl.pallas_call(
        body,
        grid=(nb,),
        in_specs=in_specs,
        out_specs=pl.BlockSpec((BM2, 2 * hidden), lambda i: (block0 + i, 0)),
        out_shape=jax.ShapeDtypeStruct((total, 2 * hidden), jnp.float32),
        input_output_aliases=io_aliases,
    )(*args)


def kernel(caption, token_embedding, positional_embedding, W, b):
    batch, seq = caption.shape
    vocab, hidden = token_embedding.shape
    b_total = batch * seq

    idx = caption.astype(jnp.int32).T.reshape(-1)
    n_chunks = b_total // CHUNK
    chunks_per_w = n_chunks // NW
    cw_slice = chunks_per_w // NSLICE
    rows_slice = NW * cw_slice * CHUNK
    idx4d = idx.reshape(NSLICE, NW, cw_slice, CHUNK)

    pa = jnp.asarray(_PA)
    pb = jnp.asarray(_PB)
    wt = W.T
    za = jnp.zeros((hidden // 2, hidden), wt.dtype)
    wa = jnp.concatenate([
        jnp.concatenate([wt[pa], za], axis=1),
        jnp.concatenate([za, wt[pa]], axis=1),
    ], axis=0)
    wb = jnp.concatenate([
        jnp.concatenate([wt[pb], za], axis=1),
        jnp.concatenate([za, wt[pb]], axis=1),
    ], axis=0)
    pos = positional_embedding[:seq]
    pos_a = jnp.concatenate([pos[:, pa], pos[:, pa]], axis=1)
    pos_a = pos_a.reshape(seq, 1, hidden)
    pos_b = jnp.concatenate([pos[:, pb], pos[:, pb]], axis=1)
    pos_b = pos_b.reshape(seq, 1, hidden)
    bias2 = jnp.concatenate([b, b]).reshape(1, 2 * hidden)

    blocks_slice = rows_slice // (2 * BM2)
    out = None
    for k in range(NSLICE):
        packed_k = _sc_gather_bf16(token_embedding, idx4d[k], hidden)
        out = _tc_linear_slice(packed_k, pos_a, pos_b, wa, wb, bias2, out,
                               k * blocks_slice, k * blocks_slice)

    return out.reshape(seq, batch, hidden).transpose(1, 0, 2)


# speedup vs baseline: 1.4391x; 1.3207x over previous
import functools

import jax
import jax.numpy as jnp
import numpy as np
from jax import lax
from jax.experimental import pallas as pl
from jax.experimental.pallas import tpu as pltpu
from jax.experimental.pallas import tpu_sc as plsc

NC = 2
NS = 16
NW = NC * NS
LANES = 16
CHUNK = 112
NSLICE = 11
BM2 = 2048

_PA = np.arange(64).reshape(4, 16)
_PA = (_PA // 16 * 32 + _PA % 16).reshape(-1)
_PB = _PA + 16


def _round_bf16_word(a_f32, b_f32):
    ua = lax.bitcast_convert_type(a_f32, jnp.int32)
    ub = lax.bitcast_convert_type(b_f32, jnp.int32)
    ra = ua + jnp.int32(0x8000)
    rb = ub + jnp.int32(0x8000)
    return ((ra >> 16) & jnp.int32(0xFFFF)) | ((rb >> 16) << 16)


def _sc_gather_bf16(table, idx3d, hidden):
    chunks_per_w = idx3d.shape[1]
    nbuf = min(4, chunks_per_w)
    b_per_w = chunks_per_w * CHUNK
    rows = NW * b_per_w
    half = CHUNK // 2
    groups = hidden // (2 * LANES)
    mesh = plsc.VectorSubcoreMesh(core_axis_name="c", subcore_axis_name="s")

    @functools.partial(
        pl.kernel,
        mesh=mesh,
        out_type=jax.ShapeDtypeStruct((rows // 2, hidden), jnp.int32),
        scratch_types=(
            [pltpu.VMEM((chunks_per_w, CHUNK), jnp.int32)]
            + [pltpu.VMEM((CHUNK, hidden), jnp.float32)] * nbuf
            + [pltpu.VMEM((half, hidden), jnp.int32)] * nbuf
            + [pltpu.SemaphoreType.DMA] * (2 * nbuf)
        ),
    )
    def gather_k(table_hbm, idx_hbm, out_hbm, idx_v, *bufs):
        fb = bufs[:nbuf]
        bb = bufs[nbuf:2 * nbuf]
        gs = bufs[2 * nbuf:3 * nbuf]
        os = bufs[3 * nbuf:]
        wid = lax.axis_index("s") * NC + lax.axis_index("c")
        base = wid * (b_per_w // 2)
        pltpu.sync_copy(idx_hbm.at[wid], idx_v)

        def gather(j, p):
            pltpu.async_copy(table_hbm.at[idx_v.at[j]], fb[p], gs[p])

        def gather_wait(p):
            pltpu.make_async_copy(
                table_hbm.at[idx_v.at[0]], fb[p], gs[p]).wait()

        def convert(p):
            fbuf, bbuf = fb[p], bb[p]

            @plsc.parallel_loop(0, half, 1, unroll=4)
            def crow(r):
                for h in range(2):
                    for g in range(groups):
                        a = fbuf[2 * r + h, pl.ds(32 * g, LANES)]
                        b = fbuf[2 * r + h, pl.ds(32 * g + LANES, LANES)]
                        bbuf[r, pl.ds((hidden // 2) * h + LANES * g, LANES)] = (
                            _round_bf16_word(a, b))

        def put(j, p):
            pltpu.async_copy(
                bb[p], out_hbm.at[pl.ds(base + j * half, half)], os[p])

        def put_wait(p):
            pltpu.make_async_copy(
                bb[p], out_hbm.at[pl.ds(base, half)], os[p]).wait()

        for j in range(nbuf):
            gather(j, j)
        for j in range(chunks_per_w):
            p = j % nbuf
            if j >= nbuf:
                put_wait(p)
            gather_wait(p)
            convert(p)
            put(j, p)
            if j + nbuf < chunks_per_w:
                gather(j + nbuf, p)
        for p in range(nbuf):
            put_wait(p)

    return gather_k(table, idx3d)


def _tc_linear_slice(x_slice, pos_a, pos_b, wa, wb, bias2, out_prev,
                     block0, pos0):
    n2, hidden = x_slice.shape
    n2_total = out_prev.shape[0] if out_prev is not None else None
    nb = n2 // BM2

    def body(x_ref, pa_ref, pb_ref, wa_ref, wb_ref, b_ref, *rest):
        o_ref = rest[-1]
        w32 = x_ref[...]
        xa = lax.bitcast_convert_type(w32 << 16, jnp.float32)
        xb = lax.bitcast_convert_type((w32 >> 16) << 16, jnp.float32)
        o_ref[...] = (
            jnp.dot(xa + pa_ref[0], wa_ref[...],
                    preferred_element_type=jnp.float32)
            + jnp.dot(xb + pb_ref[0], wb_ref[...],
                      preferred_element_type=jnp.float32)
            + b_ref[...]
        )

    in_specs = [
        pl.BlockSpec((BM2, hidden), lambda i: (i, 0)),
        pl.BlockSpec((1, 1, hidden), lambda i: (pos0 + i, 0, 0)),
        pl.BlockSpec((1, 1, hidden), lambda i: (pos0 + i, 0, 0)),
        pl.BlockSpec((hidden, 2 * hidden), lambda i: (0, 0)),
        pl.BlockSpec((hidden, 2 * hidden), lambda i: (0, 0)),
        pl.BlockSpec((1, 2 * hidden), lambda i: (0, 0)),
    ]
    args = [x_slice, pos_a, pos_b, wa, wb, bias2]
    io_aliases = {}
    if out_prev is not None:
        in_specs.append(pl.BlockSpec(memory_space=pl.ANY))
        args.append(out_prev)
        io_aliases = {6: 0}
        total = n2_total
    else:
        total = nb * BM2 * NSLICE

    return pl.pallas_call(
        body,
        grid=(nb,),
        in_specs=in_specs,
        out_specs=pl.BlockSpec((BM2, 2 * hidden), lambda i: (block0 + i, 0)),
        out_shape=jax.ShapeDtypeStruct((total, 2 * hidden), jnp.float32),
        input_output_aliases=io_aliases,
    )(*args)


def kernel(caption, token_embedding, positional_embedding, W, b):
    batch, seq = caption.shape
    vocab, hidden = token_embedding.shape
    b_total = batch * seq

    idx = caption.astype(jnp.int32).T.reshape(-1)
    n_chunks = b_total // CHUNK
    chunks_per_w = n_chunks // NW
    cw_slice = chunks_per_w // NSLICE
    rows_slice = NW * cw_slice * CHUNK
    idx4d = idx.reshape(NSLICE, NW, cw_slice, CHUNK)

    pa = jnp.asarray(_PA)
    pb = jnp.asarray(_PB)
    wt = W.T
    za = jnp.zeros((hidden // 2, hidden), wt.dtype)
    wa = jnp.concatenate([
        jnp.concatenate([wt[pa], za], axis=1),
        jnp.concatenate([za, wt[pa]], axis=1),
    ], axis=0)
    wb = jnp.concatenate([
        jnp.concatenate([wt[pb], za], axis=1),
        jnp.concatenate([za, wt[pb]], axis=1),
    ], axis=0)
    pos = positional_embedding[:seq]
    pos_a = jnp.concatenate([pos[:, pa], pos[:, pa]], axis=1)
    pos_a = pos_a.reshape(seq, 1, hidden)
    pos_b = jnp.concatenate([pos[:, pb], pos[:, pb]], axis=1)
    pos_b = pos_b.reshape(seq, 1, hidden)
    bias2 = jnp.concatenate([b, b]).reshape(1, 2 * hidden)

    blocks_slice = rows_slice // (2 * BM2)
    out = None
    for k in range(NSLICE):
        packed_k = _sc_gather_bf16(token_embedding, idx4d[k], hidden)
        out = _tc_linear_slice(packed_k, pos_a, pos_b, wa, wb, bias2, out,
                               k * blocks_slice, k * blocks_slice)

    return out.reshape(seq, batch, hidden).transpose(1, 0, 2)


# speedup vs baseline: 2.3034x; 1.6006x over previous
import functools

import jax
import jax.numpy as jnp
import numpy as np
from jax import lax
from jax.experimental import pallas as pl
from jax.experimental.pallas import tpu as pltpu
from jax.experimental.pallas import tpu_sc as plsc

NC = 2
NS = 16
NW = NC * NS
LANES = 16
CHUNK = 112
NSLICE = 11
BM2 = 2048

_PA = np.arange(64).reshape(4, 16)
_PA = (_PA // 16 * 32 + _PA % 16).reshape(-1)
_PB = _PA + 16


def _round_bf16_word(a_f32, b_f32):
    ua = lax.bitcast_convert_type(a_f32, jnp.int32)
    ub = lax.bitcast_convert_type(b_f32, jnp.int32)
    ra = ua + jnp.int32(0x8000)
    rb = ub + jnp.int32(0x8000)
    return ((ra >> 16) & jnp.int32(0xFFFF)) | ((rb >> 16) << 16)


def _sc_gather_bf16(table, idx3d, hidden):
    chunks_per_w = idx3d.shape[1]
    nbuf = min(4, chunks_per_w)
    b_per_w = chunks_per_w * CHUNK
    rows = NW * b_per_w
    half = CHUNK // 2
    groups = hidden // (2 * LANES)
    mesh = plsc.VectorSubcoreMesh(core_axis_name="c", subcore_axis_name="s")

    @functools.partial(
        pl.kernel,
        mesh=mesh,
        out_type=jax.ShapeDtypeStruct((rows // 2, hidden), jnp.int32),
        scratch_types=(
            [pltpu.VMEM((chunks_per_w, CHUNK), jnp.int32)]
            + [pltpu.VMEM((CHUNK, hidden), jnp.float32)] * nbuf
            + [pltpu.VMEM((half, hidden), jnp.int32)] * nbuf
            + [pltpu.SemaphoreType.DMA] * (2 * nbuf)
        ),
    )
    def gather_k(table_hbm, idx_hbm, out_hbm, idx_v, *bufs):
        fb = bufs[:nbuf]
        bb = bufs[nbuf:2 * nbuf]
        gs = bufs[2 * nbuf:3 * nbuf]
        os = bufs[3 * nbuf:]
        wid = lax.axis_index("s") * NC + lax.axis_index("c")
        base = wid * (b_per_w // 2)
        pltpu.sync_copy(idx_hbm.at[wid], idx_v)

        def gather(j, p):
            pltpu.async_copy(table_hbm.at[idx_v.at[j]], fb[p], gs[p])

        def gather_wait(p):
            pltpu.make_async_copy(
                table_hbm.at[idx_v.at[0]], fb[p], gs[p]).wait()

        def convert(p):
            fbuf, bbuf = fb[p], bb[p]

            @plsc.parallel_loop(0, half, 1, unroll=4)
            def crow(r):
                for h in range(2):
                    for g in range(groups):
                        a = fbuf[2 * r + h, pl.ds(32 * g, LANES)]
                        b = fbuf[2 * r + h, pl.ds(32 * g + LANES, LANES)]
                        bbuf[r, pl.ds((hidden // 2) * h + LANES * g, LANES)] = (
                            _round_bf16_word(a, b))

        def put(j, p):
            pltpu.async_copy(
                bb[p], out_hbm.at[pl.ds(base + j * half, half)], os[p])

        def put_wait(p):
            pltpu.make_async_copy(
                bb[p], out_hbm.at[pl.ds(base, half)], os[p]).wait()

        for j in range(nbuf):
            gather(j, j)
        for j in range(chunks_per_w):
            p = j % nbuf
            if j >= nbuf:
                put_wait(p)
            gather_wait(p)
            convert(p)
            put(j, p)
            if j + nbuf < chunks_per_w:
                gather(j + nbuf, p)
        for p in range(nbuf):
            put_wait(p)

    return gather_k(table, idx3d)


def _tc_linear_slice(x_slice, pos_a, pos_b, wa, wb, bias2, out_prev,
                     block0, pos0):
    n2, hidden = x_slice.shape
    nb = n2 // BM2

    def body(x_ref, pa_ref, pb_ref, wa_ref, wb_ref, b_ref, *rest):
        o_ref = rest[-1]
        w32 = x_ref[...]
        xa = lax.bitcast_convert_type(w32 << 16, jnp.float32)
        xb = lax.bitcast_convert_type((w32 >> 16) << 16, jnp.float32)
        y2 = (
            jnp.dot(xa + pa_ref[0], wa_ref[...],
                    preferred_element_type=jnp.float32)
            + jnp.dot(xb + pb_ref[0], wb_ref[...],
                      preferred_element_type=jnp.float32)
            + b_ref[...]
        )
        o_ref[:BM2, :] = y2[:, :hidden]
        o_ref[BM2:, :] = y2[:, hidden:]

    in_specs = [
        pl.BlockSpec((BM2, hidden), lambda i: (i, 0)),
        pl.BlockSpec((1, 1, hidden), lambda i: (pos0 + i, 0, 0)),
        pl.BlockSpec((1, 1, hidden), lambda i: (pos0 + i, 0, 0)),
        pl.BlockSpec((hidden, 2 * hidden), lambda i: (0, 0)),
        pl.BlockSpec((hidden, 2 * hidden), lambda i: (0, 0)),
        pl.BlockSpec((1, 2 * hidden), lambda i: (0, 0)),
    ]
    args = [x_slice, pos_a, pos_b, wa, wb, bias2]
    io_aliases = {}
    if out_prev is not None:
        in_specs.append(pl.BlockSpec(memory_space=pl.ANY))
        args.append(out_prev)
        io_aliases = {6: 0}
        total = out_prev.shape[0]
    else:
        total = 2 * BM2 * nb * NSLICE

    return pl.pallas_call(
        body,
        grid=(nb,),
        in_specs=in_specs,
        out_specs=pl.BlockSpec((2 * BM2, hidden), lambda i: (block0 + i, 0)),
        out_shape=jax.ShapeDtypeStruct((total, hidden), jnp.float32),
        input_output_aliases=io_aliases,
    )(*args)


def kernel(caption, token_embedding, positional_embedding, W, b):
    batch, seq = caption.shape
    vocab, hidden = token_embedding.shape
    b_total = batch * seq

    halfb = batch // 2
    cap_perm = jnp.stack([caption[:halfb], caption[halfb:]], axis=1)
    cap_perm = cap_perm.reshape(batch, seq)

    idx = cap_perm.astype(jnp.int32).T.reshape(-1)
    n_chunks = b_total // CHUNK
    chunks_per_w = n_chunks // NW
    cw_slice = chunks_per_w // NSLICE
    rows_slice = NW * cw_slice * CHUNK
    idx4d = idx.reshape(NSLICE, NW, cw_slice, CHUNK)

    pa = jnp.asarray(_PA)
    pb = jnp.asarray(_PB)
    wt = W.T
    za = jnp.zeros((hidden // 2, hidden), wt.dtype)
    wa = jnp.concatenate([
        jnp.concatenate([wt[pa], za], axis=1),
        jnp.concatenate([za, wt[pa]], axis=1),
    ], axis=0)
    wb = jnp.concatenate([
        jnp.concatenate([wt[pb], za], axis=1),
        jnp.concatenate([za, wt[pb]], axis=1),
    ], axis=0)
    pos = positional_embedding[:seq]
    pos_a = jnp.concatenate([pos[:, pa], pos[:, pa]], axis=1)
    pos_a = pos_a.reshape(seq, 1, hidden)
    pos_b = jnp.concatenate([pos[:, pb], pos[:, pb]], axis=1)
    pos_b = pos_b.reshape(seq, 1, hidden)
    bias2 = jnp.concatenate([b, b]).reshape(1, 2 * hidden)

    blocks_slice = rows_slice // (2 * BM2)
    out = None
    for k in range(NSLICE):
        packed_k = _sc_gather_bf16(token_embedding, idx4d[k], hidden)
        out = _tc_linear_slice(packed_k, pos_a, pos_b, wa, wb, bias2, out,
                               k * blocks_slice, k * blocks_slice)

    return out.reshape(seq, batch, hidden).transpose(1, 0, 2)
